# baseline (speedup 1.0000x reference)
import functools
import jax
import jax.numpy as jnp
from jax import lax
from jax.experimental import pallas as pl
from jax.experimental.pallas import tpu as pltpu
from jax.experimental.pallas import tpu_sc as plsc

_B, _S, _D = 4, 2048, 1024
_NC, _NS = 2, 16
_NW = _NC * _NS
_ROWS_W = _S // _NW
_CS = 8
_CHUNKS = _ROWS_W // _CS
_BP = 2
_NUNITS = _CHUNKS * (_B // _BP)
_NXS = 6
_NPS = 3
_NCOL = _D // 16

_UC = [u // (_B // _BP) for u in range(_NUNITS)]
_UB = [(u % (_B // _BP)) * _BP for u in range(_NUNITS)]


def _sc_body(x_hbm, pos_hbm, out_hbm, pos_buf, x_buf, xsems, psems):
    w = lax.axis_index("s") * _NC + lax.axis_index("c")
    base_row = w * _ROWS_W

    def issue_pos(c, ps):
        row0 = base_row + c * _CS
        return pltpu.async_copy(
            pos_hbm.at[pl.ds(row0, _CS), :],
            pos_buf.at[pl.ds(ps * _CS, _CS), :], psems.at[ps])

    def issue_in(u, xs):
        row0 = base_row + _UC[u] * _CS
        b0 = _UB[u]
        return pltpu.async_copy(
            x_hbm.at[pl.ds(b0, _BP), pl.ds(row0, _CS), :], x_buf.at[xs],
            xsems.at[2 * xs])

    def issue_out(u, xs):
        row0 = base_row + _UC[u] * _CS
        b0 = _UB[u]
        return pltpu.async_copy(
            x_buf.at[xs], out_hbm.at[pl.ds(b0, _BP), pl.ds(row0, _CS), :],
            xsems.at[2 * xs + 1])

    def compute(xs, ps):
        @plsc.parallel_loop(0, _NCOL, 1, unroll=4)
        def col_body(j):
            sl = pl.ds(j * 16, 16)
            for r in range(_CS):
                p = pos_buf[ps * _CS + r, sl]
                for b in range(_BP):
                    plsc.addupdate(x_buf.at[xs, b, r, sl], p)

    pos_d = {}
    for c in range(_NPS):
        pos_d[c] = issue_pos(c, c % _NPS)
    in_d = {}
    out_d = {}
    for u in range(_NXS):
        in_d[u] = issue_in(u, u % _NXS)

    units_per_chunk = _B // _BP
    for u in range(_NUNITS):
        c = _UC[u]
        xs = u % _NXS
        ps = c % _NPS
        if u % units_per_chunk == 0 and c in pos_d:
            pos_d.pop(c).wait()
        in_d.pop(u).wait()
        compute(xs, ps)
        out_d[u] = issue_out(u, xs)
        nxt = u + _NXS - 3
        if u >= 3 and nxt < _NUNITS:
            out_d.pop(u - 3).wait()
            in_d[nxt] = issue_in(nxt, nxt % _NXS)
        if u % units_per_chunk == units_per_chunk - 1 and c + _NPS < _CHUNKS:
            pos_d[c + _NPS] = issue_pos(c + _NPS, (c + _NPS) % _NPS)
    for u in sorted(out_d):
        out_d[u].wait()


_sc_add = functools.partial(
    pl.kernel,
    mesh=plsc.VectorSubcoreMesh(core_axis_name="c", subcore_axis_name="s"),
    out_type=jax.ShapeDtypeStruct((_B, _S, _D), jnp.float32),
    scratch_types=[
        pltpu.VMEM((_NPS * _CS, _D), jnp.float32),
        pltpu.VMEM((_NXS, _BP, _CS, _D), jnp.float32),
        pltpu.SemaphoreType.DMA((_NXS * 2,)),
        pltpu.SemaphoreType.DMA((_NPS,)),
    ],
    compiler_params=pltpu.CompilerParams(
        use_tc_tiling_on_sc=True,
        skip_device_barrier=True,
        disable_bounds_checks=True,
        disable_semaphore_checks=True,
    ),
)(_sc_body)


def kernel(x, pos_embedding):
    B, S, D = x.shape
    return _sc_add(x, pos_embedding[:S])


# speedup vs baseline: 1.0691x; 1.0691x over previous
import functools
import jax
import jax.numpy as jnp
from jax import lax
from jax.experimental import pallas as pl
from jax.experimental.pallas import tpu as pltpu
from jax.experimental.pallas import tpu_sc as plsc

_B, _S, _D = 4, 2048, 1024
_NC, _NS = 2, 16
_NW = _NC * _NS
_ROWS_W = _S // _NW
_CS = 8
_CHUNKS = _ROWS_W // _CS
_BP = 2
_NUNITS = _CHUNKS * (_B // _BP)
_NXS = 6
_NPS = 3
_NCOL = _D // 16

_UC = [u // (_B // _BP) for u in range(_NUNITS)]
_UB = [(u % (_B // _BP)) * _BP for u in range(_NUNITS)]


def _sc_body(x_hbm, pos_hbm, out_hbm, pos_buf, x_buf, xsems, psems):
    w = lax.axis_index("s") * _NC + lax.axis_index("c")
    base_row = w * _ROWS_W

    def issue_pos(c, ps):
        row0 = base_row + c * _CS
        return pltpu.async_copy(
            pos_hbm.at[pl.ds(row0, _CS), :],
            pos_buf.at[pl.ds(ps * _CS, _CS), :], psems.at[ps])

    def issue_in(u, xs):
        row0 = base_row + _UC[u] * _CS
        b0 = _UB[u]
        return pltpu.async_copy(
            x_hbm.at[pl.ds(b0, _BP), pl.ds(row0, _CS), :], x_buf.at[xs],
            xsems.at[2 * xs])

    def issue_out(u, xs):
        row0 = base_row + _UC[u] * _CS
        b0 = _UB[u]
        return pltpu.async_copy(
            x_buf.at[xs], out_hbm.at[pl.ds(b0, _BP), pl.ds(row0, _CS), :],
            xsems.at[2 * xs + 1])

    def compute(xs, ps):
        @plsc.parallel_loop(0, _NCOL, 1, unroll=1)
        def col_body(j):
            sl = pl.ds(j * 16, 16)
            for r in range(_CS):
                p = pos_buf[ps * _CS + r, sl]
                for b in range(_BP):
                    plsc.addupdate(x_buf.at[xs, b, r, sl], p)

    pos_d = {}
    for c in range(_NPS):
        pos_d[c] = issue_pos(c, c % _NPS)
    in_d = {}
    out_d = {}
    for u in range(_NXS):
        in_d[u] = issue_in(u, u % _NXS)

    units_per_chunk = _B // _BP
    for u in range(_NUNITS):
        c = _UC[u]
        xs = u % _NXS
        ps = c % _NPS
        if u % units_per_chunk == 0 and c in pos_d:
            pos_d.pop(c).wait()
        in_d.pop(u).wait()
        compute(xs, ps)
        out_d[u] = issue_out(u, xs)
        nxt = u + _NXS - 3
        if u >= 3 and nxt < _NUNITS:
            out_d.pop(u - 3).wait()
            in_d[nxt] = issue_in(nxt, nxt % _NXS)
        if u % units_per_chunk == units_per_chunk - 1 and c + _NPS < _CHUNKS:
            pos_d[c + _NPS] = issue_pos(c + _NPS, (c + _NPS) % _NPS)
    for u in sorted(out_d):
        out_d[u].wait()


_sc_add = functools.partial(
    pl.kernel,
    mesh=plsc.VectorSubcoreMesh(core_axis_name="c", subcore_axis_name="s"),
    out_type=jax.ShapeDtypeStruct((_B, _S, _D), jnp.float32),
    scratch_types=[
        pltpu.VMEM((_NPS * _CS, _D), jnp.float32),
        pltpu.VMEM((_NXS, _BP, _CS, _D), jnp.float32),
        pltpu.SemaphoreType.DMA((_NXS * 2,)),
        pltpu.SemaphoreType.DMA((_NPS,)),
    ],
    compiler_params=pltpu.CompilerParams(
        use_tc_tiling_on_sc=True,
        skip_device_barrier=True,
        disable_bounds_checks=True,
        disable_semaphore_checks=True,
    ),
)(_sc_body)


def kernel(x, pos_embedding):
    B, S, D = x.shape
    return _sc_add(x, pos_embedding[:S])


# speedup vs baseline: 1.0770x; 1.0073x over previous
import functools
import jax
import jax.numpy as jnp
from jax import lax
from jax.experimental import pallas as pl
from jax.experimental.pallas import tpu as pltpu
from jax.experimental.pallas import tpu_sc as plsc

_B, _S, _D = 4, 2048, 1024
_NC, _NS = 2, 16
_NW = _NC * _NS
_ROWS_W = _S // _NW
_CS = 8
_CHUNKS = _ROWS_W // _CS
_BP = 2
_NUNITS = _CHUNKS * (_B // _BP)
_NXS = 6
_NPS = 3
_NCOL = _D // 16

_UC = [u // (_B // _BP) for u in range(_NUNITS)]
_UB = [(u % (_B // _BP)) * _BP for u in range(_NUNITS)]


def _sc_body(x_hbm, pos_hbm, out_hbm, pos_buf, x_buf, xsems, psems):
    w = lax.axis_index("s") * _NC + lax.axis_index("c")
    base_row = w * _ROWS_W

    def issue_pos(c, ps):
        row0 = base_row + c * _CS
        return pltpu.async_copy(
            pos_hbm.at[pl.ds(row0, _CS), :],
            pos_buf.at[pl.ds(ps * _CS, _CS), :], psems.at[ps])

    def issue_in(u, xs):
        row0 = base_row + _UC[u] * _CS
        b0 = _UB[u]
        return pltpu.async_copy(
            x_hbm.at[pl.ds(b0, _BP), pl.ds(row0, _CS), :], x_buf.at[xs],
            xsems.at[2 * xs])

    def issue_out(u, xs):
        row0 = base_row + _UC[u] * _CS
        b0 = _UB[u]
        return pltpu.async_copy(
            x_buf.at[xs], out_hbm.at[pl.ds(b0, _BP), pl.ds(row0, _CS), :],
            xsems.at[2 * xs + 1])

    def compute(xs, ps):
        @plsc.parallel_loop(0, _NCOL, 1, unroll=2)
        def col_body(j):
            sl = pl.ds(j * 16, 16)
            for r in range(_CS):
                p = pos_buf[ps * _CS + r, sl]
                for b in range(_BP):
                    plsc.addupdate(x_buf.at[xs, b, r, sl], p)

    pos_d = {}
    for c in range(_NPS):
        pos_d[c] = issue_pos(c, c % _NPS)
    in_d = {}
    out_d = {}
    for u in range(_NXS):
        in_d[u] = issue_in(u, u % _NXS)

    units_per_chunk = _B // _BP
    for u in range(_NUNITS):
        c = _UC[u]
        xs = u % _NXS
        ps = c % _NPS
        if u % units_per_chunk == 0 and c in pos_d:
            pos_d.pop(c).wait()
        in_d.pop(u).wait()
        compute(xs, ps)
        out_d[u] = issue_out(u, xs)
        nxt = u + _NXS - 3
        if u >= 3 and nxt < _NUNITS:
            out_d.pop(u - 3).wait()
            in_d[nxt] = issue_in(nxt, nxt % _NXS)
        if u % units_per_chunk == units_per_chunk - 1 and c + _NPS < _CHUNKS:
            pos_d[c + _NPS] = issue_pos(c + _NPS, (c + _NPS) % _NPS)
    for u in sorted(out_d):
        out_d[u].wait()


_sc_add = functools.partial(
    pl.kernel,
    mesh=plsc.VectorSubcoreMesh(core_axis_name="c", subcore_axis_name="s"),
    out_type=jax.ShapeDtypeStruct((_B, _S, _D), jnp.float32),
    scratch_types=[
        pltpu.VMEM((_NPS * _CS, _D), jnp.float32),
        pltpu.VMEM((_NXS, _BP, _CS, _D), jnp.float32),
        pltpu.SemaphoreType.DMA((_NXS * 2,)),
        pltpu.SemaphoreType.DMA((_NPS,)),
    ],
    compiler_params=pltpu.CompilerParams(
        use_tc_tiling_on_sc=True,
        skip_device_barrier=True,
        disable_bounds_checks=True,
        disable_semaphore_checks=True,
    ),
)(_sc_body)


def kernel(x, pos_embedding):
    B, S, D = x.shape
    return _sc_add(x, pos_embedding[:S])
